# baseline (speedup 1.0000x reference)
import functools

import jax
import jax.numpy as jnp
from jax import lax
from jax.experimental import pallas as pl
from jax.experimental.pallas import tpu as pltpu
from jax.experimental.pallas import tpu_sc as plsc

_ALPHA = 0.7
_L = 16
_NC = 2
_NS = 16
_NW = _NC * _NS
_CB = 128
_CH = 256
_NQ = _CB // _L
_SC_B = 16


def _cmp_desc(x, y):
    return jnp.maximum(x, y), jnp.minimum(x, y)


def _sort4(x0, x1, x2, x3):
    a, b = _cmp_desc(x0, x1)
    c, d = _cmp_desc(x2, x3)
    s1, t = _cmp_desc(a, c)
    u, s4 = _cmp_desc(b, d)
    s2, s3 = _cmp_desc(t, u)
    return s1, s2, s3, s4


def _merge_top(acc, s):
    a1, a2, a3 = acc
    s1, s2, s3 = s
    n1 = jnp.maximum(a1, s1)
    t1 = jnp.minimum(a1, s1)
    t2 = jnp.maximum(a2, s2)
    n2 = jnp.maximum(t1, t2)
    t3 = jnp.minimum(t1, t2)
    m = jnp.minimum(a2, s2)
    n = jnp.maximum(a3, s3)
    n3 = jnp.maximum(t3, jnp.maximum(m, n))
    return n1, n2, n3


def _merge_bot(acc, s):
    a1, a2, a3 = acc
    s1, s2, s3 = s
    n1 = jnp.minimum(a1, s1)
    t1 = jnp.maximum(a1, s1)
    t2 = jnp.minimum(a2, s2)
    n2 = jnp.minimum(t1, t2)
    t3 = jnp.maximum(t1, t2)
    m = jnp.maximum(a2, s2)
    n = jnp.minimum(a3, s3)
    n3 = jnp.minimum(t3, jnp.minimum(m, n))
    return n1, n2, n3


@functools.lru_cache(maxsize=None)
def _build_sc(b_sc, c, n):
    rows = b_sc * c
    assert n % _CH == 0 and c % _CB == 0
    ncpb = n // _CH
    slabs = b_sc * (c // _CB)
    assert slabs % _NW == 0
    spw = slabs // _NW
    cpw = spw * ncpb
    spb = c // _CB
    out_per_w = spw * _CB

    mesh = plsc.VectorSubcoreMesh(core_axis_name="c", subcore_axis_name="s")

    @functools.partial(
        pl.kernel,
        out_type=jax.ShapeDtypeStruct((rows,), jnp.float32),
        mesh=mesh,
        compiler_params=pltpu.CompilerParams(
            needs_layout_passes=False, use_tc_tiling_on_sc=True),
        scratch_types=[
            pltpu.VMEM((_CH, _CB), jnp.float32),
            pltpu.VMEM((_CH, _CB), jnp.float32),
            pltpu.VMEM((ncpb * _NQ * 6 * _L,), jnp.float32),
            pltpu.VMEM((out_per_w,), jnp.float32),
            pltpu.SemaphoreType.DMA,
            pltpu.SemaphoreType.DMA,
        ],
    )
    def sc_pool(y_hbm, out_hbm, buf0, buf1, res, out_v, sem0, sem1):
        wid = lax.axis_index("s") * _NC + lax.axis_index("c")

        neg = jnp.full((_L,), -jnp.inf, jnp.float32)
        pos = jnp.full((_L,), jnp.inf, jnp.float32)

        def src(ci):
            sg = wid * spw + ci // ncpb
            chunk = ci % ncpb
            bi = sg // spb
            cb = sg % spb
            return y_hbm.at[pl.ds(bi * n + chunk * _CH, _CH),
                            pl.ds(cb * _CB, _CB)]

        def start(ci, buf, sem):
            pltpu.async_copy(src(ci), buf, sem)

        def wait(ci, buf, sem):
            pltpu.make_async_copy(src(ci), buf, sem).wait()

        def compute(buf, ci):
            chunk = ci % ncpb

            def qbody(q, _):
                cq = q * _L

                def sbody(i, cr):
                    s0 = i * 8
                    m1, m2, m3, p1, p2, p3, q1, q2, q3, r1, r2, r3 = cr
                    x0 = buf[s0, pl.ds(cq, _L)]
                    x1 = buf[s0 + 1, pl.ds(cq, _L)]
                    x2 = buf[s0 + 2, pl.ds(cq, _L)]
                    x3 = buf[s0 + 3, pl.ds(cq, _L)]
                    s1, s2, s3, s4 = _sort4(x0, x1, x2, x3)
                    m1, m2, m3 = _merge_top((m1, m2, m3), (s1, s2, s3))
                    p1, p2, p3 = _merge_bot((p1, p2, p3), (s4, s3, s2))
                    y0 = buf[s0 + 4, pl.ds(cq, _L)]
                    y1 = buf[s0 + 5, pl.ds(cq, _L)]
                    y2 = buf[s0 + 6, pl.ds(cq, _L)]
                    y3 = buf[s0 + 7, pl.ds(cq, _L)]
                    t1, t2, t3, t4 = _sort4(y0, y1, y2, y3)
                    q1, q2, q3 = _merge_top((q1, q2, q3), (t1, t2, t3))
                    r1, r2, r3 = _merge_bot((r1, r2, r3), (t4, t3, t2))
                    return (m1, m2, m3, p1, p2, p3, q1, q2, q3, r1, r2, r3)

                cr = lax.fori_loop(
                    0, _CH // 8, sbody,
                    (neg, neg, neg, pos, pos, pos,
                     neg, neg, neg, pos, pos, pos))
                m1, m2, m3, p1, p2, p3, q1, q2, q3, r1, r2, r3 = cr
                m1, m2, m3 = _merge_top((m1, m2, m3), (q1, q2, q3))
                p1, p2, p3 = _merge_bot((p1, p2, p3), (r1, r2, r3))
                base = (chunk * _NQ + q) * (6 * _L)
                res[pl.ds(base, _L)] = m1
                res[pl.ds(base + _L, _L)] = m2
                res[pl.ds(base + 2 * _L, _L)] = m3
                res[pl.ds(base + 3 * _L, _L)] = p1
                res[pl.ds(base + 4 * _L, _L)] = p2
                res[pl.ds(base + 5 * _L, _L)] = p3
                return 0

            lax.fori_loop(0, _NQ, qbody, 0)

        def finish(ci):
            si = ci // ncpb

            def qbody(q, _):
                def tri(chunk, j):
                    base = (chunk * _NQ + q) * (6 * _L) + j * _L
                    return res[pl.ds(base, _L)]

                m = (tri(0, 0), tri(0, 1), tri(0, 2))
                p = (tri(0, 3), tri(0, 4), tri(0, 5))
                for chunk in range(1, ncpb):
                    m = _merge_top(m, (tri(chunk, 0), tri(chunk, 1),
                                       tri(chunk, 2)))
                    p = _merge_bot(p, (tri(chunk, 3), tri(chunk, 4),
                                       tri(chunk, 5)))
                top = (m[0] + m[1] + m[2]) / 3.0
                bot = (p[0] + p[1] + p[2]) * (_ALPHA / 3.0)
                out_v[pl.ds(si * _CB + q * _L, _L)] = (top + bot) * 0.5
                return 0

            lax.fori_loop(0, _NQ, qbody, 0)

        start(0, buf0, sem0)

        def pair(i, carry):
            c0 = 2 * i
            start(c0 + 1, buf1, sem1)
            wait(c0, buf0, sem0)
            compute(buf0, c0)

            @pl.when(c0 % ncpb == ncpb - 1)
            def _():
                finish(c0)

            @pl.when(c0 + 2 < cpw)
            def _():
                start(c0 + 2, buf0, sem0)

            wait(c0 + 1, buf1, sem1)
            compute(buf1, c0 + 1)

            @pl.when((c0 + 1) % ncpb == ncpb - 1)
            def _():
                finish(c0 + 1)

            return carry

        lax.fori_loop(0, cpw // 2, pair, 0)
        pltpu.sync_copy(out_v, out_hbm.at[pl.ds(wid * out_per_w, out_per_w)])

    return sc_pool


@functools.lru_cache(maxsize=None)
def _build_tc(b0, nb, c, n):
    assert n % 32 == 0 and c % _CB == 0

    def tc_pool(x_ref, o_ref):
        neg = jnp.full((8, _CB), -jnp.inf, jnp.float32)
        pos = jnp.full((8, _CB), jnp.inf, jnp.float32)

        def body(i, cr):
            s0 = pl.multiple_of(i * 32, 32)
            m1, m2, m3, p1, p2, p3 = cr
            x0 = x_ref[pl.ds(s0, 8), :]
            x1 = x_ref[pl.ds(s0 + 8, 8), :]
            x2 = x_ref[pl.ds(s0 + 16, 8), :]
            x3 = x_ref[pl.ds(s0 + 24, 8), :]
            s1, s2, s3, s4 = _sort4(x0, x1, x2, x3)
            m1, m2, m3 = _merge_top((m1, m2, m3), (s1, s2, s3))
            p1, p2, p3 = _merge_bot((p1, p2, p3), (s4, s3, s2))
            return (m1, m2, m3, p1, p2, p3)

        m1, m2, m3, p1, p2, p3 = lax.fori_loop(
            0, n // 32, body, (neg, neg, neg, pos, pos, pos))
        m = (m1, m2, m3)
        p = (p1, p2, p3)
        for k in (4, 2, 1):
            m = _merge_top(tuple(v[0:k] for v in m),
                           tuple(v[k:2 * k] for v in m))
            p = _merge_bot(tuple(v[0:k] for v in p),
                           tuple(v[k:2 * k] for v in p))
        top = (m[0] + m[1] + m[2]) / 3.0
        bot = (p[0] + p[1] + p[2]) * (_ALPHA / 3.0)
        o_ref[...] = ((top + bot) * 0.5).reshape(1, 1, _CB)

    return pl.pallas_call(
        tc_pool,
        grid=(nb, c // _CB),
        in_specs=[pl.BlockSpec((n, _CB), lambda i, j: (b0 + i, j))],
        out_specs=pl.BlockSpec((1, 1, _CB), lambda i, j: (i, 0, j)),
        out_shape=jax.ShapeDtypeStruct((nb, 1, c), jnp.float32),
    )


def kernel(input):
    b, c, h, w = input.shape
    n = h * w
    y = input.transpose(0, 2, 3, 1).reshape(b * n, c)
    sc_out = _build_sc(_SC_B, c, n)(y)
    tc_out = _build_tc(_SC_B, b - _SC_B, c, n)(y)
    out = jnp.concatenate([sc_out, tc_out.reshape((b - _SC_B) * c)])
    return out.reshape(b, c)


# speedup vs baseline: 1.0605x; 1.0605x over previous
import functools

import jax
import jax.numpy as jnp
from jax import lax
from jax.experimental import pallas as pl
from jax.experimental.pallas import tpu as pltpu
from jax.experimental.pallas import tpu_sc as plsc

_ALPHA = 0.7
_L = 16
_NC = 2
_NS = 16
_NW = _NC * _NS
_CB = 128
_CH = 256
_NQ = _CB // _L
_SC_B = 16


def _cmp_desc(x, y):
    return jnp.maximum(x, y), jnp.minimum(x, y)


def _sort4(x0, x1, x2, x3):
    a, b = _cmp_desc(x0, x1)
    c, d = _cmp_desc(x2, x3)
    s1, t = _cmp_desc(a, c)
    u, s4 = _cmp_desc(b, d)
    s2, s3 = _cmp_desc(t, u)
    return s1, s2, s3, s4


def _merge_top(acc, s):
    a1, a2, a3 = acc
    s1, s2, s3 = s
    n1 = jnp.maximum(a1, s1)
    t1 = jnp.minimum(a1, s1)
    t2 = jnp.maximum(a2, s2)
    n2 = jnp.maximum(t1, t2)
    t3 = jnp.minimum(t1, t2)
    m = jnp.minimum(a2, s2)
    n = jnp.maximum(a3, s3)
    n3 = jnp.maximum(t3, jnp.maximum(m, n))
    return n1, n2, n3


def _merge_bot(acc, s):
    a1, a2, a3 = acc
    s1, s2, s3 = s
    n1 = jnp.minimum(a1, s1)
    t1 = jnp.maximum(a1, s1)
    t2 = jnp.minimum(a2, s2)
    n2 = jnp.minimum(t1, t2)
    t3 = jnp.maximum(t1, t2)
    m = jnp.maximum(a2, s2)
    n = jnp.minimum(a3, s3)
    n3 = jnp.minimum(t3, jnp.minimum(m, n))
    return n1, n2, n3


@functools.lru_cache(maxsize=None)
def _build_sc(b_sc, c, n):
    rows = b_sc * c
    assert n % _CH == 0 and c % _CB == 0
    ncpb = n // _CH
    slabs = b_sc * (c // _CB)
    assert slabs % _NW == 0
    spw = slabs // _NW
    cpw = spw * ncpb
    spb = c // _CB
    out_per_w = spw * _CB

    mesh = plsc.VectorSubcoreMesh(core_axis_name="c", subcore_axis_name="s")

    @functools.partial(
        pl.kernel,
        out_type=jax.ShapeDtypeStruct((rows,), jnp.float32),
        mesh=mesh,
        compiler_params=pltpu.CompilerParams(
            needs_layout_passes=False, use_tc_tiling_on_sc=True),
        scratch_types=[
            pltpu.VMEM((_CH, _CB), jnp.float32),
            pltpu.VMEM((_CH, _CB), jnp.float32),
            pltpu.VMEM((ncpb * _NQ * 6 * _L,), jnp.float32),
            pltpu.VMEM((out_per_w,), jnp.float32),
            pltpu.SemaphoreType.DMA,
            pltpu.SemaphoreType.DMA,
        ],
    )
    def sc_pool(y_hbm, out_hbm, buf0, buf1, res, out_v, sem0, sem1):
        wid = lax.axis_index("s") * _NC + lax.axis_index("c")

        neg = jnp.full((_L,), -jnp.inf, jnp.float32)
        pos = jnp.full((_L,), jnp.inf, jnp.float32)

        def src(ci):
            sg = wid * spw + ci // ncpb
            chunk = ci % ncpb
            bi = sg // spb
            cb = sg % spb
            return y_hbm.at[pl.ds(bi * n + chunk * _CH, _CH),
                            pl.ds(cb * _CB, _CB)]

        def start(ci, buf, sem):
            pltpu.async_copy(src(ci), buf, sem)

        def wait(ci, buf, sem):
            pltpu.make_async_copy(src(ci), buf, sem).wait()

        def compute(buf, ci):
            chunk = ci % ncpb

            def qbody(q, _):
                cq = q * _L

                def sbody(i, cr):
                    s0 = i * 8
                    m1, m2, m3, p1, p2, p3, q1, q2, q3, r1, r2, r3 = cr
                    x0 = buf[s0, pl.ds(cq, _L)]
                    x1 = buf[s0 + 1, pl.ds(cq, _L)]
                    x2 = buf[s0 + 2, pl.ds(cq, _L)]
                    x3 = buf[s0 + 3, pl.ds(cq, _L)]
                    s1, s2, s3, s4 = _sort4(x0, x1, x2, x3)
                    m1, m2, m3 = _merge_top((m1, m2, m3), (s1, s2, s3))
                    p1, p2, p3 = _merge_bot((p1, p2, p3), (s4, s3, s2))
                    y0 = buf[s0 + 4, pl.ds(cq, _L)]
                    y1 = buf[s0 + 5, pl.ds(cq, _L)]
                    y2 = buf[s0 + 6, pl.ds(cq, _L)]
                    y3 = buf[s0 + 7, pl.ds(cq, _L)]
                    t1, t2, t3, t4 = _sort4(y0, y1, y2, y3)
                    q1, q2, q3 = _merge_top((q1, q2, q3), (t1, t2, t3))
                    r1, r2, r3 = _merge_bot((r1, r2, r3), (t4, t3, t2))
                    return (m1, m2, m3, p1, p2, p3, q1, q2, q3, r1, r2, r3)

                cr = lax.fori_loop(
                    0, _CH // 8, sbody,
                    (neg, neg, neg, pos, pos, pos,
                     neg, neg, neg, pos, pos, pos))
                m1, m2, m3, p1, p2, p3, q1, q2, q3, r1, r2, r3 = cr
                m1, m2, m3 = _merge_top((m1, m2, m3), (q1, q2, q3))
                p1, p2, p3 = _merge_bot((p1, p2, p3), (r1, r2, r3))
                base = (chunk * _NQ + q) * (6 * _L)
                res[pl.ds(base, _L)] = m1
                res[pl.ds(base + _L, _L)] = m2
                res[pl.ds(base + 2 * _L, _L)] = m3
                res[pl.ds(base + 3 * _L, _L)] = p1
                res[pl.ds(base + 4 * _L, _L)] = p2
                res[pl.ds(base + 5 * _L, _L)] = p3
                return 0

            lax.fori_loop(0, _NQ, qbody, 0)

        def finish(ci):
            si = ci // ncpb

            def qbody(q, _):
                def tri(chunk, j):
                    base = (chunk * _NQ + q) * (6 * _L) + j * _L
                    return res[pl.ds(base, _L)]

                m = (tri(0, 0), tri(0, 1), tri(0, 2))
                p = (tri(0, 3), tri(0, 4), tri(0, 5))
                for chunk in range(1, ncpb):
                    m = _merge_top(m, (tri(chunk, 0), tri(chunk, 1),
                                       tri(chunk, 2)))
                    p = _merge_bot(p, (tri(chunk, 3), tri(chunk, 4),
                                       tri(chunk, 5)))
                top = (m[0] + m[1] + m[2]) / 3.0
                bot = (p[0] + p[1] + p[2]) * (_ALPHA / 3.0)
                out_v[pl.ds(si * _CB + q * _L, _L)] = (top + bot) * 0.5
                return 0

            lax.fori_loop(0, _NQ, qbody, 0)

        start(0, buf0, sem0)

        def pair(i, carry):
            c0 = 2 * i
            start(c0 + 1, buf1, sem1)
            wait(c0, buf0, sem0)
            compute(buf0, c0)

            @pl.when(c0 % ncpb == ncpb - 1)
            def _():
                finish(c0)

            @pl.when(c0 + 2 < cpw)
            def _():
                start(c0 + 2, buf0, sem0)

            wait(c0 + 1, buf1, sem1)
            compute(buf1, c0 + 1)

            @pl.when((c0 + 1) % ncpb == ncpb - 1)
            def _():
                finish(c0 + 1)

            return carry

        lax.fori_loop(0, cpw // 2, pair, 0)
        pltpu.sync_copy(out_v, out_hbm.at[pl.ds(wid * out_per_w, out_per_w)])

    return sc_pool


@functools.lru_cache(maxsize=None)
def _build_tc(b0, nb, c, n):
    assert n % 32 == 0 and c % _CB == 0

    def tc_pool(x_ref, o_ref):
        neg = jnp.full((8, _CB), -jnp.inf, jnp.float32)
        pos = jnp.full((8, _CB), jnp.inf, jnp.float32)

        def quad(s0):
            x0 = x_ref[pl.ds(s0, 8), :]
            x1 = x_ref[pl.ds(s0 + 8, 8), :]
            x2 = x_ref[pl.ds(s0 + 16, 8), :]
            x3 = x_ref[pl.ds(s0 + 24, 8), :]
            return _sort4(x0, x1, x2, x3)

        def body(i, cr):
            s0 = pl.multiple_of(i * 64, 64)
            m1, m2, m3, p1, p2, p3, q1, q2, q3, r1, r2, r3 = cr
            s1, s2, s3, s4 = quad(s0)
            m1, m2, m3 = _merge_top((m1, m2, m3), (s1, s2, s3))
            p1, p2, p3 = _merge_bot((p1, p2, p3), (s4, s3, s2))
            t1, t2, t3, t4 = quad(s0 + 32)
            q1, q2, q3 = _merge_top((q1, q2, q3), (t1, t2, t3))
            r1, r2, r3 = _merge_bot((r1, r2, r3), (t4, t3, t2))
            return (m1, m2, m3, p1, p2, p3, q1, q2, q3, r1, r2, r3)

        m1, m2, m3, p1, p2, p3, q1, q2, q3, r1, r2, r3 = lax.fori_loop(
            0, n // 64, body,
            (neg, neg, neg, pos, pos, pos, neg, neg, neg, pos, pos, pos))
        m = _merge_top((m1, m2, m3), (q1, q2, q3))
        p = _merge_bot((p1, p2, p3), (r1, r2, r3))
        for k in (4, 2, 1):
            m = _merge_top(tuple(v[0:k] for v in m),
                           tuple(v[k:2 * k] for v in m))
            p = _merge_bot(tuple(v[0:k] for v in p),
                           tuple(v[k:2 * k] for v in p))
        top = (m[0] + m[1] + m[2]) / 3.0
        bot = (p[0] + p[1] + p[2]) * (_ALPHA / 3.0)
        o_ref[...] = ((top + bot) * 0.5).reshape(1, 1, _CB)

    return pl.pallas_call(
        tc_pool,
        grid=(nb, c // _CB),
        in_specs=[pl.BlockSpec((n, _CB), lambda i, j: (b0 + i, j))],
        out_specs=pl.BlockSpec((1, 1, _CB), lambda i, j: (i, 0, j)),
        out_shape=jax.ShapeDtypeStruct((nb, 1, c), jnp.float32),
    )


def kernel(input):
    b, c, h, w = input.shape
    n = h * w
    y = input.transpose(0, 2, 3, 1).reshape(b * n, c)
    sc_out = _build_sc(_SC_B, c, n)(y)
    tc_out = _build_tc(_SC_B, b - _SC_B, c, n)(y)
    out = jnp.concatenate([sc_out, tc_out.reshape((b - _SC_B) * c)])
    return out.reshape(b, c)


# speedup vs baseline: 1.0695x; 1.0084x over previous
import functools

import jax
import jax.numpy as jnp
from jax import lax
from jax.experimental import pallas as pl
from jax.experimental.pallas import tpu as pltpu
from jax.experimental.pallas import tpu_sc as plsc

_ALPHA = 0.7
_L = 16
_NC = 2
_NS = 16
_NW = _NC * _NS
_CB = 128
_CH = 256
_NQ = _CB // _L
_SC_B = 16


def _cmp_desc(x, y):
    return jnp.maximum(x, y), jnp.minimum(x, y)


def _sort4(x0, x1, x2, x3):
    a, b = _cmp_desc(x0, x1)
    c, d = _cmp_desc(x2, x3)
    s1, t = _cmp_desc(a, c)
    u, s4 = _cmp_desc(b, d)
    s2, s3 = _cmp_desc(t, u)
    return s1, s2, s3, s4


def _merge_top(acc, s):
    a1, a2, a3 = acc
    s1, s2, s3 = s
    n1 = jnp.maximum(a1, s1)
    t1 = jnp.minimum(a1, s1)
    t2 = jnp.maximum(a2, s2)
    n2 = jnp.maximum(t1, t2)
    t3 = jnp.minimum(t1, t2)
    m = jnp.minimum(a2, s2)
    n = jnp.maximum(a3, s3)
    n3 = jnp.maximum(t3, jnp.maximum(m, n))
    return n1, n2, n3


def _merge_bot(acc, s):
    a1, a2, a3 = acc
    s1, s2, s3 = s
    n1 = jnp.minimum(a1, s1)
    t1 = jnp.maximum(a1, s1)
    t2 = jnp.minimum(a2, s2)
    n2 = jnp.minimum(t1, t2)
    t3 = jnp.maximum(t1, t2)
    m = jnp.maximum(a2, s2)
    n = jnp.minimum(a3, s3)
    n3 = jnp.minimum(t3, jnp.minimum(m, n))
    return n1, n2, n3


@functools.lru_cache(maxsize=None)
def _build_sc(b_sc, c, n):
    rows = b_sc * c
    assert n % _CH == 0 and c % _CB == 0
    ncpb = n // _CH
    slabs = b_sc * (c // _CB)
    assert slabs % _NW == 0
    spw = slabs // _NW
    cpw = spw * ncpb
    spb = c // _CB
    out_per_w = spw * _CB

    mesh = plsc.VectorSubcoreMesh(core_axis_name="c", subcore_axis_name="s")

    @functools.partial(
        pl.kernel,
        out_type=jax.ShapeDtypeStruct((rows,), jnp.float32),
        mesh=mesh,
        compiler_params=pltpu.CompilerParams(
            needs_layout_passes=False, use_tc_tiling_on_sc=True),
        scratch_types=[
            pltpu.VMEM((_CH, _CB), jnp.float32),
            pltpu.VMEM((_CH, _CB), jnp.float32),
            pltpu.VMEM((ncpb * _NQ * 6 * _L,), jnp.float32),
            pltpu.VMEM((out_per_w,), jnp.float32),
            pltpu.SemaphoreType.DMA,
            pltpu.SemaphoreType.DMA,
        ],
    )
    def sc_pool(y_hbm, out_hbm, buf0, buf1, res, out_v, sem0, sem1):
        wid = lax.axis_index("s") * _NC + lax.axis_index("c")

        neg = jnp.full((_L,), -jnp.inf, jnp.float32)
        pos = jnp.full((_L,), jnp.inf, jnp.float32)

        def src(ci):
            sg = wid * spw + ci // ncpb
            chunk = ci % ncpb
            bi = sg // spb
            cb = sg % spb
            return y_hbm.at[pl.ds(bi * n + chunk * _CH, _CH),
                            pl.ds(cb * _CB, _CB)]

        def start(ci, buf, sem):
            pltpu.async_copy(src(ci), buf, sem)

        def wait(ci, buf, sem):
            pltpu.make_async_copy(src(ci), buf, sem).wait()

        def compute(buf, ci):
            chunk = ci % ncpb

            def qbody(q, _):
                cq = q * _L

                def sbody(i, cr):
                    s0 = i * 8
                    m1, m2, m3, p1, p2, p3, q1, q2, q3, r1, r2, r3 = cr
                    x0 = buf[s0, pl.ds(cq, _L)]
                    x1 = buf[s0 + 1, pl.ds(cq, _L)]
                    x2 = buf[s0 + 2, pl.ds(cq, _L)]
                    x3 = buf[s0 + 3, pl.ds(cq, _L)]
                    s1, s2, s3, s4 = _sort4(x0, x1, x2, x3)
                    m1, m2, m3 = _merge_top((m1, m2, m3), (s1, s2, s3))
                    p1, p2, p3 = _merge_bot((p1, p2, p3), (s4, s3, s2))
                    y0 = buf[s0 + 4, pl.ds(cq, _L)]
                    y1 = buf[s0 + 5, pl.ds(cq, _L)]
                    y2 = buf[s0 + 6, pl.ds(cq, _L)]
                    y3 = buf[s0 + 7, pl.ds(cq, _L)]
                    t1, t2, t3, t4 = _sort4(y0, y1, y2, y3)
                    q1, q2, q3 = _merge_top((q1, q2, q3), (t1, t2, t3))
                    r1, r2, r3 = _merge_bot((r1, r2, r3), (t4, t3, t2))
                    return (m1, m2, m3, p1, p2, p3, q1, q2, q3, r1, r2, r3)

                cr = lax.fori_loop(
                    0, _CH // 8, sbody,
                    (neg, neg, neg, pos, pos, pos,
                     neg, neg, neg, pos, pos, pos))
                m1, m2, m3, p1, p2, p3, q1, q2, q3, r1, r2, r3 = cr
                m1, m2, m3 = _merge_top((m1, m2, m3), (q1, q2, q3))
                p1, p2, p3 = _merge_bot((p1, p2, p3), (r1, r2, r3))
                base = (chunk * _NQ + q) * (6 * _L)
                res[pl.ds(base, _L)] = m1
                res[pl.ds(base + _L, _L)] = m2
                res[pl.ds(base + 2 * _L, _L)] = m3
                res[pl.ds(base + 3 * _L, _L)] = p1
                res[pl.ds(base + 4 * _L, _L)] = p2
                res[pl.ds(base + 5 * _L, _L)] = p3
                return 0

            lax.fori_loop(0, _NQ, qbody, 0)

        def finish(ci):
            si = ci // ncpb

            def qbody(q, _):
                def tri(chunk, j):
                    base = (chunk * _NQ + q) * (6 * _L) + j * _L
                    return res[pl.ds(base, _L)]

                m = (tri(0, 0), tri(0, 1), tri(0, 2))
                p = (tri(0, 3), tri(0, 4), tri(0, 5))
                for chunk in range(1, ncpb):
                    m = _merge_top(m, (tri(chunk, 0), tri(chunk, 1),
                                       tri(chunk, 2)))
                    p = _merge_bot(p, (tri(chunk, 3), tri(chunk, 4),
                                       tri(chunk, 5)))
                top = (m[0] + m[1] + m[2]) / 3.0
                bot = (p[0] + p[1] + p[2]) * (_ALPHA / 3.0)
                out_v[pl.ds(si * _CB + q * _L, _L)] = (top + bot) * 0.5
                return 0

            lax.fori_loop(0, _NQ, qbody, 0)

        start(0, buf0, sem0)

        def pair(i, carry):
            c0 = 2 * i
            start(c0 + 1, buf1, sem1)
            wait(c0, buf0, sem0)
            compute(buf0, c0)

            @pl.when(c0 % ncpb == ncpb - 1)
            def _():
                finish(c0)

            @pl.when(c0 + 2 < cpw)
            def _():
                start(c0 + 2, buf0, sem0)

            wait(c0 + 1, buf1, sem1)
            compute(buf1, c0 + 1)

            @pl.when((c0 + 1) % ncpb == ncpb - 1)
            def _():
                finish(c0 + 1)

            return carry

        lax.fori_loop(0, cpw // 2, pair, 0)
        pltpu.sync_copy(out_v, out_hbm.at[pl.ds(wid * out_per_w, out_per_w)])

    return sc_pool


@functools.lru_cache(maxsize=None)
def _build_tc(b0, nb, c, n):
    assert n % 32 == 0 and c % _CB == 0

    def tc_pool(x_ref, o_ref):
        neg = jnp.full((8, _CB), -jnp.inf, jnp.float32)
        pos = jnp.full((8, _CB), jnp.inf, jnp.float32)

        def quad(s0):
            x0 = x_ref[pl.ds(s0, 8), :]
            x1 = x_ref[pl.ds(s0 + 8, 8), :]
            x2 = x_ref[pl.ds(s0 + 16, 8), :]
            x3 = x_ref[pl.ds(s0 + 24, 8), :]
            return _sort4(x0, x1, x2, x3)

        def body(i, cr):
            s0 = pl.multiple_of(i * 128, 128)
            ms = list(cr[:12])
            ps = list(cr[12:])
            for k in range(4):
                s1, s2, s3, s4 = quad(s0 + 32 * k)
                ms[3 * k:3 * k + 3] = _merge_top(
                    tuple(ms[3 * k:3 * k + 3]), (s1, s2, s3))
                ps[3 * k:3 * k + 3] = _merge_bot(
                    tuple(ps[3 * k:3 * k + 3]), (s4, s3, s2))
            return tuple(ms) + tuple(ps)

        cr = lax.fori_loop(
            0, n // 128, body, (neg,) * 12 + (pos,) * 12)
        ms, ps = cr[:12], cr[12:]
        m = tuple(ms[0:3])
        p = tuple(ps[0:3])
        for k in range(1, 4):
            m = _merge_top(m, tuple(ms[3 * k:3 * k + 3]))
            p = _merge_bot(p, tuple(ps[3 * k:3 * k + 3]))
        for k in (4, 2, 1):
            m = _merge_top(tuple(v[0:k] for v in m),
                           tuple(v[k:2 * k] for v in m))
            p = _merge_bot(tuple(v[0:k] for v in p),
                           tuple(v[k:2 * k] for v in p))
        top = (m[0] + m[1] + m[2]) / 3.0
        bot = (p[0] + p[1] + p[2]) * (_ALPHA / 3.0)
        o_ref[...] = ((top + bot) * 0.5).reshape(1, 1, _CB)

    return pl.pallas_call(
        tc_pool,
        grid=(nb, c // _CB),
        in_specs=[pl.BlockSpec((n, _CB), lambda i, j: (b0 + i, j))],
        out_specs=pl.BlockSpec((1, 1, _CB), lambda i, j: (i, 0, j)),
        out_shape=jax.ShapeDtypeStruct((nb, 1, c), jnp.float32),
    )


def kernel(input):
    b, c, h, w = input.shape
    n = h * w
    y = input.transpose(0, 2, 3, 1).reshape(b * n, c)
    sc_out = _build_sc(_SC_B, c, n)(y)
    tc_out = _build_tc(_SC_B, b - _SC_B, c, n)(y)
    out = jnp.concatenate([sc_out, tc_out.reshape((b - _SC_B) * c)])
    return out.reshape(b, c)


# speedup vs baseline: 2.1417x; 2.0025x over previous
import functools

import jax
import jax.numpy as jnp
from jax import lax
from jax.experimental import pallas as pl
from jax.experimental.pallas import tpu as pltpu
from jax.experimental.pallas import tpu_sc as plsc

_ALPHA = 0.7
_L = 16
_NC = 2
_NS = 16
_NW = _NC * _NS
_CB = 128
_CH = 256
_NQ = _CB // _L
_SC_B = 16


def _cmp_desc(x, y):
    return jnp.maximum(x, y), jnp.minimum(x, y)


def _sort4(x0, x1, x2, x3):
    a, b = _cmp_desc(x0, x1)
    c, d = _cmp_desc(x2, x3)
    s1, t = _cmp_desc(a, c)
    u, s4 = _cmp_desc(b, d)
    s2, s3 = _cmp_desc(t, u)
    return s1, s2, s3, s4


def _merge_top(acc, s):
    a1, a2, a3 = acc
    s1, s2, s3 = s
    n1 = jnp.maximum(a1, s1)
    t1 = jnp.minimum(a1, s1)
    t2 = jnp.maximum(a2, s2)
    n2 = jnp.maximum(t1, t2)
    t3 = jnp.minimum(t1, t2)
    m = jnp.minimum(a2, s2)
    n = jnp.maximum(a3, s3)
    n3 = jnp.maximum(t3, jnp.maximum(m, n))
    return n1, n2, n3


def _merge_bot(acc, s):
    a1, a2, a3 = acc
    s1, s2, s3 = s
    n1 = jnp.minimum(a1, s1)
    t1 = jnp.maximum(a1, s1)
    t2 = jnp.minimum(a2, s2)
    n2 = jnp.minimum(t1, t2)
    t3 = jnp.maximum(t1, t2)
    m = jnp.maximum(a2, s2)
    n = jnp.minimum(a3, s3)
    n3 = jnp.minimum(t3, jnp.minimum(m, n))
    return n1, n2, n3


@functools.lru_cache(maxsize=None)
def _build_sc(b_sc, c, n):
    rows = b_sc * c
    assert n % _CH == 0 and c % _CB == 0
    ncpb = n // _CH
    slabs = b_sc * (c // _CB)
    assert slabs % _NW == 0
    spw = slabs // _NW
    cpw = spw * ncpb
    spb = c // _CB
    out_per_w = spw * _CB

    mesh = plsc.VectorSubcoreMesh(core_axis_name="c", subcore_axis_name="s")

    @functools.partial(
        pl.kernel,
        out_type=jax.ShapeDtypeStruct((rows,), jnp.float32),
        mesh=mesh,
        compiler_params=pltpu.CompilerParams(
            needs_layout_passes=False, use_tc_tiling_on_sc=True),
        scratch_types=[
            pltpu.VMEM((_CH, _CB), jnp.float32),
            pltpu.VMEM((_CH, _CB), jnp.float32),
            pltpu.VMEM((ncpb * _NQ * 6 * _L,), jnp.float32),
            pltpu.VMEM((out_per_w,), jnp.float32),
            pltpu.SemaphoreType.DMA,
            pltpu.SemaphoreType.DMA,
        ],
    )
    def sc_pool(y_hbm, out_hbm, buf0, buf1, res, out_v, sem0, sem1):
        wid = lax.axis_index("s") * _NC + lax.axis_index("c")

        neg = jnp.full((_L,), -jnp.inf, jnp.float32)
        pos = jnp.full((_L,), jnp.inf, jnp.float32)

        def src(ci):
            sg = wid * spw + ci // ncpb
            chunk = ci % ncpb
            bi = sg // spb
            cb = sg % spb
            return y_hbm.at[pl.ds(bi * n + chunk * _CH, _CH),
                            pl.ds(cb * _CB, _CB)]

        def start(ci, buf, sem):
            pltpu.async_copy(src(ci), buf, sem)

        def wait(ci, buf, sem):
            pltpu.make_async_copy(src(ci), buf, sem).wait()

        def compute(buf, ci):
            chunk = ci % ncpb

            def qbody(q, _):
                cq = q * _L

                def sbody(i, cr):
                    s0 = i * 8
                    m1, m2, m3, p1, p2, p3, q1, q2, q3, r1, r2, r3 = cr
                    x0 = buf[s0, pl.ds(cq, _L)]
                    x1 = buf[s0 + 1, pl.ds(cq, _L)]
                    x2 = buf[s0 + 2, pl.ds(cq, _L)]
                    x3 = buf[s0 + 3, pl.ds(cq, _L)]
                    s1, s2, s3, s4 = _sort4(x0, x1, x2, x3)
                    m1, m2, m3 = _merge_top((m1, m2, m3), (s1, s2, s3))
                    p1, p2, p3 = _merge_bot((p1, p2, p3), (s4, s3, s2))
                    y0 = buf[s0 + 4, pl.ds(cq, _L)]
                    y1 = buf[s0 + 5, pl.ds(cq, _L)]
                    y2 = buf[s0 + 6, pl.ds(cq, _L)]
                    y3 = buf[s0 + 7, pl.ds(cq, _L)]
                    t1, t2, t3, t4 = _sort4(y0, y1, y2, y3)
                    q1, q2, q3 = _merge_top((q1, q2, q3), (t1, t2, t3))
                    r1, r2, r3 = _merge_bot((r1, r2, r3), (t4, t3, t2))
                    return (m1, m2, m3, p1, p2, p3, q1, q2, q3, r1, r2, r3)

                cr = lax.fori_loop(
                    0, _CH // 8, sbody,
                    (neg, neg, neg, pos, pos, pos,
                     neg, neg, neg, pos, pos, pos))
                m1, m2, m3, p1, p2, p3, q1, q2, q3, r1, r2, r3 = cr
                m1, m2, m3 = _merge_top((m1, m2, m3), (q1, q2, q3))
                p1, p2, p3 = _merge_bot((p1, p2, p3), (r1, r2, r3))
                base = (chunk * _NQ + q) * (6 * _L)
                res[pl.ds(base, _L)] = m1
                res[pl.ds(base + _L, _L)] = m2
                res[pl.ds(base + 2 * _L, _L)] = m3
                res[pl.ds(base + 3 * _L, _L)] = p1
                res[pl.ds(base + 4 * _L, _L)] = p2
                res[pl.ds(base + 5 * _L, _L)] = p3
                return 0

            lax.fori_loop(0, _NQ, qbody, 0)

        def finish(ci):
            si = ci // ncpb

            def qbody(q, _):
                def tri(chunk, j):
                    base = (chunk * _NQ + q) * (6 * _L) + j * _L
                    return res[pl.ds(base, _L)]

                m = (tri(0, 0), tri(0, 1), tri(0, 2))
                p = (tri(0, 3), tri(0, 4), tri(0, 5))
                for chunk in range(1, ncpb):
                    m = _merge_top(m, (tri(chunk, 0), tri(chunk, 1),
                                       tri(chunk, 2)))
                    p = _merge_bot(p, (tri(chunk, 3), tri(chunk, 4),
                                       tri(chunk, 5)))
                top = (m[0] + m[1] + m[2]) / 3.0
                bot = (p[0] + p[1] + p[2]) * (_ALPHA / 3.0)
                out_v[pl.ds(si * _CB + q * _L, _L)] = (top + bot) * 0.5
                return 0

            lax.fori_loop(0, _NQ, qbody, 0)

        start(0, buf0, sem0)

        def pair(i, carry):
            c0 = 2 * i
            start(c0 + 1, buf1, sem1)
            wait(c0, buf0, sem0)
            compute(buf0, c0)

            @pl.when(c0 % ncpb == ncpb - 1)
            def _():
                finish(c0)

            @pl.when(c0 + 2 < cpw)
            def _():
                start(c0 + 2, buf0, sem0)

            wait(c0 + 1, buf1, sem1)
            compute(buf1, c0 + 1)

            @pl.when((c0 + 1) % ncpb == ncpb - 1)
            def _():
                finish(c0 + 1)

            return carry

        lax.fori_loop(0, cpw // 2, pair, 0)
        pltpu.sync_copy(out_v, out_hbm.at[pl.ds(wid * out_per_w, out_per_w)])

    return sc_pool


@functools.lru_cache(maxsize=None)
def _build_tc(b0, nb, c, n):
    assert n % 32 == 0 and c % _CB == 0

    def tc_pool(x_ref, o_ref):
        neg = jnp.full((8, c), -jnp.inf, jnp.float32)
        pos = jnp.full((8, c), jnp.inf, jnp.float32)

        def quad(s0):
            x0 = x_ref[pl.ds(s0, 8), :]
            x1 = x_ref[pl.ds(s0 + 8, 8), :]
            x2 = x_ref[pl.ds(s0 + 16, 8), :]
            x3 = x_ref[pl.ds(s0 + 24, 8), :]
            return _sort4(x0, x1, x2, x3)

        def body(i, cr):
            s0 = pl.multiple_of(i * 128, 128)
            ms = list(cr[:12])
            ps = list(cr[12:])
            for k in range(4):
                s1, s2, s3, s4 = quad(s0 + 32 * k)
                ms[3 * k:3 * k + 3] = _merge_top(
                    tuple(ms[3 * k:3 * k + 3]), (s1, s2, s3))
                ps[3 * k:3 * k + 3] = _merge_bot(
                    tuple(ps[3 * k:3 * k + 3]), (s4, s3, s2))
            return tuple(ms) + tuple(ps)

        cr = lax.fori_loop(
            0, n // 128, body, (neg,) * 12 + (pos,) * 12)
        ms, ps = cr[:12], cr[12:]
        m = tuple(ms[0:3])
        p = tuple(ps[0:3])
        for k in range(1, 4):
            m = _merge_top(m, tuple(ms[3 * k:3 * k + 3]))
            p = _merge_bot(p, tuple(ps[3 * k:3 * k + 3]))
        for k in (4, 2, 1):
            m = _merge_top(tuple(v[0:k] for v in m),
                           tuple(v[k:2 * k] for v in m))
            p = _merge_bot(tuple(v[0:k] for v in p),
                           tuple(v[k:2 * k] for v in p))
        top = (m[0] + m[1] + m[2]) / 3.0
        bot = (p[0] + p[1] + p[2]) * (_ALPHA / 3.0)
        o_ref[...] = ((top + bot) * 0.5).reshape(1, 1, c)

    return pl.pallas_call(
        tc_pool,
        grid=(nb,),
        in_specs=[pl.BlockSpec((n, c), lambda i: (b0 + i, 0))],
        out_specs=pl.BlockSpec((1, 1, c), lambda i: (i, 0, 0)),
        out_shape=jax.ShapeDtypeStruct((nb, 1, c), jnp.float32),
    )


def kernel(input):
    b, c, h, w = input.shape
    n = h * w
    y = input.transpose(0, 2, 3, 1).reshape(b * n, c)
    sc_out = _build_sc(_SC_B, c, n)(y)
    tc_out = _build_tc(_SC_B, b - _SC_B, c, n)(y)
    out = jnp.concatenate([sc_out, tc_out.reshape((b - _SC_B) * c)])
    return out.reshape(b, c)


# speedup vs baseline: 2.3341x; 1.0898x over previous
import functools

import jax
import jax.numpy as jnp
from jax import lax
from jax.experimental import pallas as pl
from jax.experimental.pallas import tpu as pltpu
from jax.experimental.pallas import tpu_sc as plsc

_ALPHA = 0.7
_L = 16
_NC = 2
_NS = 16
_NW = _NC * _NS
_CB = 128
_CH = 256
_NQ = _CB // _L
_SC_B = 32


def _cmp_desc(x, y):
    return jnp.maximum(x, y), jnp.minimum(x, y)


def _sort4(x0, x1, x2, x3):
    a, b = _cmp_desc(x0, x1)
    c, d = _cmp_desc(x2, x3)
    s1, t = _cmp_desc(a, c)
    u, s4 = _cmp_desc(b, d)
    s2, s3 = _cmp_desc(t, u)
    return s1, s2, s3, s4


def _merge_top(acc, s):
    a1, a2, a3 = acc
    s1, s2, s3 = s
    n1 = jnp.maximum(a1, s1)
    t1 = jnp.minimum(a1, s1)
    t2 = jnp.maximum(a2, s2)
    n2 = jnp.maximum(t1, t2)
    t3 = jnp.minimum(t1, t2)
    m = jnp.minimum(a2, s2)
    n = jnp.maximum(a3, s3)
    n3 = jnp.maximum(t3, jnp.maximum(m, n))
    return n1, n2, n3


def _merge_bot(acc, s):
    a1, a2, a3 = acc
    s1, s2, s3 = s
    n1 = jnp.minimum(a1, s1)
    t1 = jnp.maximum(a1, s1)
    t2 = jnp.minimum(a2, s2)
    n2 = jnp.minimum(t1, t2)
    t3 = jnp.maximum(t1, t2)
    m = jnp.maximum(a2, s2)
    n = jnp.minimum(a3, s3)
    n3 = jnp.minimum(t3, jnp.minimum(m, n))
    return n1, n2, n3


@functools.lru_cache(maxsize=None)
def _build_sc(b_sc, c, n):
    rows = b_sc * c
    assert n % _CH == 0 and c % _CB == 0
    ncpb = n // _CH
    slabs = b_sc * (c // _CB)
    assert slabs % _NW == 0
    spw = slabs // _NW
    cpw = spw * ncpb
    spb = c // _CB
    out_per_w = spw * _CB

    mesh = plsc.VectorSubcoreMesh(core_axis_name="c", subcore_axis_name="s")

    @functools.partial(
        pl.kernel,
        out_type=jax.ShapeDtypeStruct((rows,), jnp.float32),
        mesh=mesh,
        compiler_params=pltpu.CompilerParams(
            needs_layout_passes=False, use_tc_tiling_on_sc=True),
        scratch_types=[
            pltpu.VMEM((_CH, _CB), jnp.float32),
            pltpu.VMEM((_CH, _CB), jnp.float32),
            pltpu.VMEM((ncpb * _NQ * 6 * _L,), jnp.float32),
            pltpu.VMEM((out_per_w,), jnp.float32),
            pltpu.SemaphoreType.DMA,
            pltpu.SemaphoreType.DMA,
        ],
    )
    def sc_pool(y_hbm, out_hbm, buf0, buf1, res, out_v, sem0, sem1):
        wid = lax.axis_index("s") * _NC + lax.axis_index("c")

        neg = jnp.full((_L,), -jnp.inf, jnp.float32)
        pos = jnp.full((_L,), jnp.inf, jnp.float32)

        def src(ci):
            sg = wid * spw + ci // ncpb
            chunk = ci % ncpb
            bi = sg // spb
            cb = sg % spb
            return y_hbm.at[pl.ds(bi * n + chunk * _CH, _CH),
                            pl.ds(cb * _CB, _CB)]

        def start(ci, buf, sem):
            pltpu.async_copy(src(ci), buf, sem)

        def wait(ci, buf, sem):
            pltpu.make_async_copy(src(ci), buf, sem).wait()

        def compute(buf, ci):
            chunk = ci % ncpb

            def qbody(q, _):
                cq = q * _L

                def sbody(i, cr):
                    s0 = i * 8
                    m1, m2, m3, p1, p2, p3, q1, q2, q3, r1, r2, r3 = cr
                    x0 = buf[s0, pl.ds(cq, _L)]
                    x1 = buf[s0 + 1, pl.ds(cq, _L)]
                    x2 = buf[s0 + 2, pl.ds(cq, _L)]
                    x3 = buf[s0 + 3, pl.ds(cq, _L)]
                    s1, s2, s3, s4 = _sort4(x0, x1, x2, x3)
                    m1, m2, m3 = _merge_top((m1, m2, m3), (s1, s2, s3))
                    p1, p2, p3 = _merge_bot((p1, p2, p3), (s4, s3, s2))
                    y0 = buf[s0 + 4, pl.ds(cq, _L)]
                    y1 = buf[s0 + 5, pl.ds(cq, _L)]
                    y2 = buf[s0 + 6, pl.ds(cq, _L)]
                    y3 = buf[s0 + 7, pl.ds(cq, _L)]
                    t1, t2, t3, t4 = _sort4(y0, y1, y2, y3)
                    q1, q2, q3 = _merge_top((q1, q2, q3), (t1, t2, t3))
                    r1, r2, r3 = _merge_bot((r1, r2, r3), (t4, t3, t2))
                    return (m1, m2, m3, p1, p2, p3, q1, q2, q3, r1, r2, r3)

                cr = lax.fori_loop(
                    0, _CH // 8, sbody,
                    (neg, neg, neg, pos, pos, pos,
                     neg, neg, neg, pos, pos, pos))
                m1, m2, m3, p1, p2, p3, q1, q2, q3, r1, r2, r3 = cr
                m1, m2, m3 = _merge_top((m1, m2, m3), (q1, q2, q3))
                p1, p2, p3 = _merge_bot((p1, p2, p3), (r1, r2, r3))
                base = (chunk * _NQ + q) * (6 * _L)
                res[pl.ds(base, _L)] = m1
                res[pl.ds(base + _L, _L)] = m2
                res[pl.ds(base + 2 * _L, _L)] = m3
                res[pl.ds(base + 3 * _L, _L)] = p1
                res[pl.ds(base + 4 * _L, _L)] = p2
                res[pl.ds(base + 5 * _L, _L)] = p3
                return 0

            lax.fori_loop(0, _NQ, qbody, 0)

        def finish(ci):
            si = ci // ncpb

            def qbody(q, _):
                def tri(chunk, j):
                    base = (chunk * _NQ + q) * (6 * _L) + j * _L
                    return res[pl.ds(base, _L)]

                m = (tri(0, 0), tri(0, 1), tri(0, 2))
                p = (tri(0, 3), tri(0, 4), tri(0, 5))
                for chunk in range(1, ncpb):
                    m = _merge_top(m, (tri(chunk, 0), tri(chunk, 1),
                                       tri(chunk, 2)))
                    p = _merge_bot(p, (tri(chunk, 3), tri(chunk, 4),
                                       tri(chunk, 5)))
                top = (m[0] + m[1] + m[2]) / 3.0
                bot = (p[0] + p[1] + p[2]) * (_ALPHA / 3.0)
                out_v[pl.ds(si * _CB + q * _L, _L)] = (top + bot) * 0.5
                return 0

            lax.fori_loop(0, _NQ, qbody, 0)

        start(0, buf0, sem0)

        def pair(i, carry):
            c0 = 2 * i
            start(c0 + 1, buf1, sem1)
            wait(c0, buf0, sem0)
            compute(buf0, c0)

            @pl.when(c0 % ncpb == ncpb - 1)
            def _():
                finish(c0)

            @pl.when(c0 + 2 < cpw)
            def _():
                start(c0 + 2, buf0, sem0)

            wait(c0 + 1, buf1, sem1)
            compute(buf1, c0 + 1)

            @pl.when((c0 + 1) % ncpb == ncpb - 1)
            def _():
                finish(c0 + 1)

            return carry

        lax.fori_loop(0, cpw // 2, pair, 0)
        pltpu.sync_copy(out_v, out_hbm.at[pl.ds(wid * out_per_w, out_per_w)])

    return sc_pool


@functools.lru_cache(maxsize=None)
def _build_tc(b0, nb, c, n):
    assert n % 32 == 0 and c % _CB == 0

    def tc_pool(x_ref, o_ref):
        neg = jnp.full((8, c), -jnp.inf, jnp.float32)
        pos = jnp.full((8, c), jnp.inf, jnp.float32)

        def quad(s0):
            x0 = x_ref[pl.ds(s0, 8), :]
            x1 = x_ref[pl.ds(s0 + 8, 8), :]
            x2 = x_ref[pl.ds(s0 + 16, 8), :]
            x3 = x_ref[pl.ds(s0 + 24, 8), :]
            return _sort4(x0, x1, x2, x3)

        def body(i, cr):
            s0 = pl.multiple_of(i * 128, 128)
            ms = list(cr[:12])
            ps = list(cr[12:])
            for k in range(4):
                s1, s2, s3, s4 = quad(s0 + 32 * k)
                ms[3 * k:3 * k + 3] = _merge_top(
                    tuple(ms[3 * k:3 * k + 3]), (s1, s2, s3))
                ps[3 * k:3 * k + 3] = _merge_bot(
                    tuple(ps[3 * k:3 * k + 3]), (s4, s3, s2))
            return tuple(ms) + tuple(ps)

        cr = lax.fori_loop(
            0, n // 128, body, (neg,) * 12 + (pos,) * 12)
        ms, ps = cr[:12], cr[12:]
        m = tuple(ms[0:3])
        p = tuple(ps[0:3])
        for k in range(1, 4):
            m = _merge_top(m, tuple(ms[3 * k:3 * k + 3]))
            p = _merge_bot(p, tuple(ps[3 * k:3 * k + 3]))
        for k in (4, 2, 1):
            m = _merge_top(tuple(v[0:k] for v in m),
                           tuple(v[k:2 * k] for v in m))
            p = _merge_bot(tuple(v[0:k] for v in p),
                           tuple(v[k:2 * k] for v in p))
        top = (m[0] + m[1] + m[2]) / 3.0
        bot = (p[0] + p[1] + p[2]) * (_ALPHA / 3.0)
        o_ref[...] = ((top + bot) * 0.5).reshape(1, 1, c)

    return pl.pallas_call(
        tc_pool,
        grid=(nb,),
        in_specs=[pl.BlockSpec((n, c), lambda i: (b0 + i, 0))],
        out_specs=pl.BlockSpec((1, 1, c), lambda i: (i, 0, 0)),
        out_shape=jax.ShapeDtypeStruct((nb, 1, c), jnp.float32),
    )


def kernel(input):
    b, c, h, w = input.shape
    n = h * w
    y = input.transpose(0, 2, 3, 1).reshape(b * n, c)
    sc_out = _build_sc(_SC_B, c, n)(y)
    tc_out = _build_tc(_SC_B, b - _SC_B, c, n)(y)
    out = jnp.concatenate([sc_out, tc_out.reshape((b - _SC_B) * c)])
    return out.reshape(b, c)
